# baseline (speedup 1.0000x reference)
import functools

import jax
import jax.numpy as jnp
from jax import lax
from jax.experimental import pallas as pl
from jax.experimental.pallas import tpu as pltpu
from jax.experimental.pallas import tpu_sc as plsc

D = 64
B = 16384
NC = 1000000
CBLK = 65536
NBLK = (NC + CBLK - 1) // CBLK
GROWS = NBLK * (CBLK // 4)
MBLK = 2048
MASK16 = -65536


def _round_bf16_bits(x):
    bits = lax.bitcast_convert_type(x, jnp.int32)
    rne = lax.add(
        lax.add(bits, jnp.int32(0x7FFF)),
        lax.bitwise_and(lax.shift_right_logical(bits, 16), jnp.int32(1)),
    )
    return lax.bitwise_and(rne, jnp.int32(MASK16))


def _tc_table_w1(table_t, w1cat):
    q = CBLK // 4

    def body(t_ref, w_ref, o_ref):
        t = t_ref[...].astype(jnp.bfloat16)
        w = w_ref[...]
        dn = (((0,), (0,)), ((), ()))
        ta = jnp.concatenate([t[:, :q], t[:, q : 2 * q]], axis=0)
        tb = jnp.concatenate([t[:, 2 * q : 3 * q], t[:, 3 * q :]], axis=0)
        a = lax.dot_general(ta, w, dn, preferred_element_type=jnp.float32)
        b = lax.dot_general(tb, w, dn, preferred_element_type=jnp.float32)
        packed = lax.bitwise_or(
            lax.shift_right_logical(_round_bf16_bits(a), 16),
            _round_bf16_bits(b),
        )
        o_ref[...] = lax.bitcast_convert_type(packed, jnp.float32)

    return pl.pallas_call(
        body,
        grid=(NBLK,),
        compiler_params=pltpu.CompilerParams(vmem_limit_bytes=100 * 1024 * 1024),
        in_specs=[
            pl.BlockSpec((D, CBLK), lambda i: (0, i)),
            pl.BlockSpec((2 * D, 2 * D), lambda i: (0, 0)),
        ],
        out_specs=pl.BlockSpec((CBLK // 4, 2 * D), lambda i: (i, 0)),
        out_shape=jax.ShapeDtypeStruct((GROWS, 2 * D), jnp.float32),
    )(table_t, w1cat)


def _sc_gather(g, idx):
    info = plsc.get_sparse_core_info()
    nw = info.num_cores * info.num_subcores
    b_per_w = B // nw
    K = 16
    inner = 8
    outer = b_per_w // (inner * K)
    mesh = plsc.VectorSubcoreMesh(core_axis_name="c", subcore_axis_name="s")

    @functools.partial(
        pl.kernel,
        mesh=mesh,
        out_type=jax.ShapeDtypeStruct((B, 2 * D), jnp.float32),
        scratch_types=[
            pltpu.VMEM((b_per_w,), jnp.int32),
            pltpu.VMEM((b_per_w, 2 * D), jnp.float32),
            pltpu.SemaphoreType.DMA,
        ],
    )
    def k(g_hbm, idx_hbm, out_hbm, idx_v, rows_v, sem):
        wid = lax.axis_index("s") * info.num_cores + lax.axis_index("c")
        base = wid * b_per_w
        pltpu.sync_copy(idx_hbm.at[pl.ds(base, b_per_w)], idx_v)

        def chunk(ci, carry):
            cbase = ci * inner * K
            copies = []
            for j in range(inner):
                vec = idx_v[pl.ds(cbase + j * K, K)]
                row = ((vec >> 16) << 14) | (vec & 16383)
                copies.append(
                    pltpu.async_copy(
                        g_hbm.at[row],
                        rows_v.at[pl.ds(cbase + j * K, K)],
                        sem,
                    )
                )
            for cp in copies:
                cp.wait()
            return carry

        lax.fori_loop(0, outer, chunk, 0)
        pltpu.sync_copy(rows_v, out_hbm.at[pl.ds(base, b_per_w)])

    return k(g, idx.reshape(B))


def _tc_mlp(p, c1, b1, W2, b2):
    def body(p_ref, c_ref, b1_ref, w2_ref, b2_ref, o_ref):
        bits = lax.bitcast_convert_type(p_ref[...], jnp.int32)
        cv = lax.transpose(c_ref[...], (1, 0))
        hw = ((cv >> 15) & 1) == 1
        lh = ((cv >> 14) & 1) == 1
        vbits = jnp.where(
            hw, lax.bitwise_and(bits, jnp.int32(MASK16)), lax.shift_left(bits, 16)
        )
        v = lax.bitcast_convert_type(vbits, jnp.float32)
        sel = jnp.where(lh, v[:, D:], v[:, :D])
        h = sel + b1_ref[...]
        h = h * jax.nn.sigmoid(h)
        o = jnp.dot(h, w2_ref[...], preferred_element_type=jnp.float32)
        o_ref[...] = o + b2_ref[...]

    return pl.pallas_call(
        body,
        grid=(B // MBLK,),
        in_specs=[
            pl.BlockSpec((MBLK, 2 * D), lambda i: (i, 0)),
            pl.BlockSpec((1, MBLK), lambda i: (0, i)),
            pl.BlockSpec((1, D), lambda i: (0, 0)),
            pl.BlockSpec((D, D), lambda i: (0, 0)),
            pl.BlockSpec((1, D), lambda i: (0, 0)),
        ],
        out_specs=pl.BlockSpec((MBLK, D), lambda i: (i, 0)),
        out_shape=jax.ShapeDtypeStruct((B, D), jnp.float32),
    )(p, c1, b1.reshape(1, D), W2, b2.reshape(1, D))


def kernel(c, emb_table, W1, b1, W2, b2):
    zeros = jnp.zeros((D, D), jnp.float32)
    w1cat = jnp.concatenate(
        [jnp.concatenate([W1, zeros], axis=1),
         jnp.concatenate([zeros, W1], axis=1)], axis=0
    ).astype(jnp.bfloat16)
    g = _tc_table_w1(emb_table.T, w1cat)
    p = _sc_gather(g, c)
    return _tc_mlp(p, c.reshape(1, B), b1, W2, b2)


# speedup vs baseline: 1.0226x; 1.0226x over previous
import functools

import jax
import jax.numpy as jnp
from jax import lax
from jax.experimental import pallas as pl
from jax.experimental.pallas import tpu as pltpu
from jax.experimental.pallas import tpu_sc as plsc

D = 64
B = 16384
NC = 1000000
CBLK = 32768
NBLK = (NC + CBLK - 1) // CBLK
GROWS = NBLK * (CBLK // 4)
MBLK = 2048
MASK16 = -65536


def _round_bf16_bits(x):
    bits = lax.bitcast_convert_type(x, jnp.int32)
    rne = lax.add(
        lax.add(bits, jnp.int32(0x7FFF)),
        lax.bitwise_and(lax.shift_right_logical(bits, 16), jnp.int32(1)),
    )
    return lax.bitwise_and(rne, jnp.int32(MASK16))


def _tc_table_w1(table_t, w1cat):
    q = CBLK // 4

    def body(t_ref, w_ref, o_ref):
        t = t_ref[...].astype(jnp.bfloat16)
        w = w_ref[...]
        dn = (((0,), (0,)), ((), ()))
        ta = jnp.concatenate([t[:, :q], t[:, q : 2 * q]], axis=0)
        tb = jnp.concatenate([t[:, 2 * q : 3 * q], t[:, 3 * q :]], axis=0)
        a = lax.dot_general(ta, w, dn, preferred_element_type=jnp.float32)
        b = lax.dot_general(tb, w, dn, preferred_element_type=jnp.float32)
        packed = lax.bitwise_or(
            lax.shift_right_logical(_round_bf16_bits(a), 16),
            _round_bf16_bits(b),
        )
        o_ref[...] = lax.bitcast_convert_type(packed, jnp.float32)

    return pl.pallas_call(
        body,
        grid=(NBLK,),
        compiler_params=pltpu.CompilerParams(vmem_limit_bytes=100 * 1024 * 1024),
        in_specs=[
            pl.BlockSpec((D, CBLK), lambda i: (0, i)),
            pl.BlockSpec((2 * D, 2 * D), lambda i: (0, 0)),
        ],
        out_specs=pl.BlockSpec((CBLK // 4, 2 * D), lambda i: (i, 0)),
        out_shape=jax.ShapeDtypeStruct((GROWS, 2 * D), jnp.float32),
    )(table_t, w1cat)


def _sc_gather(g, idx):
    info = plsc.get_sparse_core_info()
    nw = info.num_cores * info.num_subcores
    b_per_w = B // nw
    K = 16
    inner = 8
    outer = b_per_w // (inner * K)
    mesh = plsc.VectorSubcoreMesh(core_axis_name="c", subcore_axis_name="s")

    @functools.partial(
        pl.kernel,
        mesh=mesh,
        out_type=jax.ShapeDtypeStruct((B, 2 * D), jnp.float32),
        scratch_types=[
            pltpu.VMEM((b_per_w,), jnp.int32),
            pltpu.VMEM((b_per_w, 2 * D), jnp.float32),
            pltpu.SemaphoreType.DMA,
        ],
    )
    def k(g_hbm, idx_hbm, out_hbm, idx_v, rows_v, sem):
        wid = lax.axis_index("s") * info.num_cores + lax.axis_index("c")
        base = wid * b_per_w
        pltpu.sync_copy(idx_hbm.at[pl.ds(base, b_per_w)], idx_v)

        def chunk(ci, carry):
            cbase = ci * inner * K
            copies = []
            for j in range(inner):
                vec = idx_v[pl.ds(cbase + j * K, K)]
                row = ((vec >> 15) << 13) | (vec & 8191)
                copies.append(
                    pltpu.async_copy(
                        g_hbm.at[row],
                        rows_v.at[pl.ds(cbase + j * K, K)],
                        sem,
                    )
                )
            for cp in copies:
                cp.wait()
            return carry

        lax.fori_loop(0, outer, chunk, 0)
        pltpu.sync_copy(rows_v, out_hbm.at[pl.ds(base, b_per_w)])

    return k(g, idx.reshape(B))


def _tc_mlp(p, c1, b1, W2, b2):
    def body(p_ref, c_ref, b1_ref, w2_ref, b2_ref, o_ref):
        bits = lax.bitcast_convert_type(p_ref[...], jnp.int32)
        cv = lax.transpose(c_ref[...], (1, 0))
        hw = ((cv >> 14) & 1) == 1
        lh = ((cv >> 13) & 1) == 1
        vbits = jnp.where(
            hw, lax.bitwise_and(bits, jnp.int32(MASK16)), lax.shift_left(bits, 16)
        )
        v = lax.bitcast_convert_type(vbits, jnp.float32)
        sel = jnp.where(lh, v[:, D:], v[:, :D])
        h = sel + b1_ref[...]
        h = h * jax.nn.sigmoid(h)
        o = jnp.dot(h, w2_ref[...], preferred_element_type=jnp.float32)
        o_ref[...] = o + b2_ref[...]

    return pl.pallas_call(
        body,
        grid=(B // MBLK,),
        in_specs=[
            pl.BlockSpec((MBLK, 2 * D), lambda i: (i, 0)),
            pl.BlockSpec((1, MBLK), lambda i: (0, i)),
            pl.BlockSpec((1, D), lambda i: (0, 0)),
            pl.BlockSpec((D, D), lambda i: (0, 0)),
            pl.BlockSpec((1, D), lambda i: (0, 0)),
        ],
        out_specs=pl.BlockSpec((MBLK, D), lambda i: (i, 0)),
        out_shape=jax.ShapeDtypeStruct((B, D), jnp.float32),
    )(p, c1, b1.reshape(1, D), W2, b2.reshape(1, D))


def kernel(c, emb_table, W1, b1, W2, b2):
    zeros = jnp.zeros((D, D), jnp.float32)
    w1cat = jnp.concatenate(
        [jnp.concatenate([W1, zeros], axis=1),
         jnp.concatenate([zeros, W1], axis=1)], axis=0
    ).astype(jnp.bfloat16)
    g = _tc_table_w1(emb_table.T, w1cat)
    p = _sc_gather(g, c)
    return _tc_mlp(p, c.reshape(1, B), b1, W2, b2)


# speedup vs baseline: 1.0715x; 1.0479x over previous
import functools

import jax
import jax.numpy as jnp
from jax import lax
from jax.experimental import pallas as pl
from jax.experimental.pallas import tpu as pltpu
from jax.experimental.pallas import tpu_sc as plsc

D = 64
B = 16384
NC = 1000000
CBLK = 32768
NBLK = (NC + CBLK - 1) // CBLK
GROWS = NBLK * (CBLK // 4)
MBLK = 2048
MASK16 = -65536


def _round_bf16_bits(x):
    bits = lax.bitcast_convert_type(x, jnp.int32)
    rne = lax.add(
        lax.add(bits, jnp.int32(0x7FFF)),
        lax.bitwise_and(lax.shift_right_logical(bits, 16), jnp.int32(1)),
    )
    return lax.bitwise_and(rne, jnp.int32(MASK16))


def _tc_table_w1(table_t, w1cat):
    q = CBLK // 4

    def body(t_ref, w_ref, o_ref):
        t = t_ref[...].astype(jnp.bfloat16)
        w = w_ref[...]
        dn = (((0,), (0,)), ((), ()))
        ta = jnp.concatenate([t[:, :q], t[:, q : 2 * q]], axis=0)
        tb = jnp.concatenate([t[:, 2 * q : 3 * q], t[:, 3 * q :]], axis=0)
        a = lax.dot_general(ta, w, dn, preferred_element_type=jnp.float32)
        b = lax.dot_general(tb, w, dn, preferred_element_type=jnp.float32)
        packed = lax.bitwise_or(
            lax.shift_right_logical(_round_bf16_bits(a), 16),
            _round_bf16_bits(b),
        )
        o_ref[...] = lax.bitcast_convert_type(packed, jnp.float32)

    return pl.pallas_call(
        body,
        grid=(NBLK,),
        compiler_params=pltpu.CompilerParams(vmem_limit_bytes=100 * 1024 * 1024),
        in_specs=[
            pl.BlockSpec((D, CBLK), lambda i: (0, i)),
            pl.BlockSpec((2 * D, 2 * D), lambda i: (0, 0)),
        ],
        out_specs=pl.BlockSpec((CBLK // 4, 2 * D), lambda i: (i, 0)),
        out_shape=jax.ShapeDtypeStruct((GROWS, 2 * D), jnp.float32),
    )(table_t, w1cat)


def _sc_gather(g, idx):
    info = plsc.get_sparse_core_info()
    nw = info.num_cores * info.num_subcores
    b_per_w = B // nw
    K = 16
    inner = 16
    outer = b_per_w // (inner * K)
    mesh = plsc.VectorSubcoreMesh(core_axis_name="c", subcore_axis_name="s")

    @functools.partial(
        pl.kernel,
        mesh=mesh,
        out_type=jax.ShapeDtypeStruct((B, 2 * D), jnp.float32),
        scratch_types=[
            pltpu.VMEM((b_per_w,), jnp.int32),
            pltpu.VMEM((b_per_w, 2 * D), jnp.float32),
            pltpu.SemaphoreType.DMA,
        ],
    )
    def k(g_hbm, idx_hbm, out_hbm, idx_v, rows_v, sem):
        wid = lax.axis_index("s") * info.num_cores + lax.axis_index("c")
        base = wid * b_per_w
        pltpu.sync_copy(idx_hbm.at[pl.ds(base, b_per_w)], idx_v)

        def chunk(ci, carry):
            cbase = ci * inner * K
            copies = []
            for j in range(inner):
                vec = idx_v[pl.ds(cbase + j * K, K)]
                row = ((vec >> 15) << 13) | (vec & 8191)
                copies.append(
                    pltpu.async_copy(
                        g_hbm.at[row],
                        rows_v.at[pl.ds(cbase + j * K, K)],
                        sem,
                    )
                )
            for cp in copies:
                cp.wait()
            return carry

        lax.fori_loop(0, outer, chunk, 0)
        pltpu.sync_copy(rows_v, out_hbm.at[pl.ds(base, b_per_w)])

    return k(g, idx.reshape(B))


def _tc_mlp(p, c1, b1, W2, b2):
    def body(p_ref, c_ref, b1_ref, w2_ref, b2_ref, o_ref):
        bits = lax.bitcast_convert_type(p_ref[...], jnp.int32)
        cv = lax.transpose(c_ref[...], (1, 0))
        hw = ((cv >> 14) & 1) == 1
        lh = ((cv >> 13) & 1) == 1
        vbits = jnp.where(
            hw, lax.bitwise_and(bits, jnp.int32(MASK16)), lax.shift_left(bits, 16)
        )
        v = lax.bitcast_convert_type(vbits, jnp.float32)
        sel = jnp.where(lh, v[:, D:], v[:, :D])
        h = sel + b1_ref[...]
        h = h * jax.nn.sigmoid(h)
        o = lax.dot_general(
            w2_ref[...], h, (((0,), (1,)), ((), ())),
            preferred_element_type=jnp.float32,
        )
        o_ref[...] = o + b2_ref[...]

    return pl.pallas_call(
        body,
        grid=(B // MBLK,),
        in_specs=[
            pl.BlockSpec((MBLK, 2 * D), lambda i: (i, 0)),
            pl.BlockSpec((1, MBLK), lambda i: (0, i)),
            pl.BlockSpec((1, D), lambda i: (0, 0)),
            pl.BlockSpec((D, D), lambda i: (0, 0)),
            pl.BlockSpec((D, 1), lambda i: (0, 0)),
        ],
        out_specs=pl.BlockSpec((D, MBLK), lambda i: (0, i)),
        out_shape=jax.ShapeDtypeStruct((D, B), jnp.float32),
    )(p, c1, b1.reshape(1, D), W2, b2.reshape(D, 1))


def kernel(c, emb_table, W1, b1, W2, b2):
    zeros = jnp.zeros((D, D), jnp.float32)
    w1cat = jnp.concatenate(
        [jnp.concatenate([W1, zeros], axis=1),
         jnp.concatenate([zeros, W1], axis=1)], axis=0
    ).astype(jnp.bfloat16)
    g = _tc_table_w1(emb_table.T, w1cat)
    p = _sc_gather(g, c)
    return _tc_mlp(p, c.reshape(1, B), b1, W2, b2).T
